# baseline (speedup 1.0000x reference)
import jax
import jax.numpy as jnp
import numpy as np
from jax.experimental import pallas as pl
from jax.experimental.pallas import tpu as pltpu

N = 4096
KOFF = 63
BLOCK_ROWS = 512
INT32_MIN = np.int32(-2147483648)
INT32_MAX = np.int32(2147483647)
SIGN_LO = np.int32(0x7FFFFFFF)


def _f32_to_key(x):
    bits = jax.lax.bitcast_convert_type(x, jnp.int32)
    return bits ^ (jax.lax.shift_right_arithmetic(bits, 31) & SIGN_LO)


def _key_to_f32(k):
    bits = k ^ (jax.lax.shift_right_arithmetic(k, 31) & SIGN_LO)
    return jax.lax.bitcast_convert_type(bits, jnp.float32)


def _block_kernel(x_ref, o_ref):
    i = pl.program_id(0)
    x = x_ref[...]
    r = x.shape[0]
    key = _f32_to_key(x)
    row = jax.lax.broadcasted_iota(jnp.int32, (r, N), 0) + i * r
    col = jax.lax.broadcasted_iota(jnp.int32, (r, N), 1)
    is_diag = row == col
    key = jnp.where(is_diag, INT32_MIN, key)

    def count_ge(t):
        return jnp.sum(jnp.where(key >= t, jnp.int32(1), jnp.int32(0)),
                       axis=1, keepdims=True)

    def bisect(lo, hi):
        return lo + jax.lax.shift_right_logical(hi - lo, 1)

    def step(lo, hi, clo, chi, mid):
        cnt = count_ge(mid)
        ge = cnt >= KOFF
        return (jnp.where(ge, mid, lo), jnp.where(ge, hi, mid),
                jnp.where(ge, cnt, clo), jnp.where(ge, chi, cnt))

    rowmax = jnp.max(key, axis=1, keepdims=True)
    lo = jnp.full((r, 1), INT32_MIN)
    hi = rowmax + 1
    clo = jnp.full((r, 1), np.int32(N))
    chi = jnp.zeros((r, 1), jnp.int32)
    lo, hi, clo, chi = step(lo, hi, clo, chi, bisect(lo, hi))
    lo, hi, clo, chi = step(lo, hi, clo, chi, bisect(lo, hi))

    def cond(carry):
        lo, hi, clo, chi, j = carry
        return jnp.logical_and(j < 40, jnp.any(hi - lo > 1))

    def body(carry):
        lo, hi, clo, chi, j = carry
        flo = _key_to_f32(lo)
        fhi = _key_to_f32(hi)
        frac = ((clo - KOFF).astype(jnp.float32)
                / jnp.maximum(clo - chi, 1).astype(jnp.float32))
        t = flo + (fhi - flo) * frac
        mid = jnp.clip(_f32_to_key(t), lo + 1, hi - 1)
        lo, hi, clo, chi = step(lo, hi, clo, chi, mid)
        mid = bisect(lo, hi)
        mid = jnp.clip(mid, lo, hi - 1)
        lo, hi, clo, chi = step(lo, hi, clo, chi, mid)
        return lo, hi, clo, chi, j + 1

    lo, hi, clo, chi, _ = jax.lax.while_loop(
        cond, body, (lo, hi, clo, chi, jnp.int32(0)))
    thr = lo

    a = jax.nn.sigmoid(x)
    keep = jnp.logical_or(key >= thr, is_diag)
    a = jnp.where(is_diag, jnp.float32(1.0), a)
    o_ref[...] = jnp.where(keep, a, jnp.float32(0.0))


@jax.jit
def kernel(logits):
    grid = (N // BLOCK_ROWS,)
    return pl.pallas_call(
        _block_kernel,
        grid=grid,
        in_specs=[pl.BlockSpec((BLOCK_ROWS, N), lambda i: (i, 0))],
        out_specs=pl.BlockSpec((BLOCK_ROWS, N), lambda i: (i, 0)),
        out_shape=jax.ShapeDtypeStruct((N, N), jnp.float32),
        compiler_params=pltpu.CompilerParams(
            dimension_semantics=("arbitrary",),
        ),
    )(logits)


# speedup vs baseline: 2.2618x; 2.2618x over previous
import jax
import jax.numpy as jnp
import numpy as np
from jax.experimental import pallas as pl
from jax.experimental.pallas import tpu as pltpu

N = 4096
KOFF = 63
BLOCK_ROWS = 512
INT32_MIN = np.int32(-2147483648)
SIGN_LO = np.int32(0x7FFFFFFF)
ONE16 = np.int16(1)
ZERO16 = np.int16(0)


def _block_kernel(x_ref, o_ref):
    i = pl.program_id(0)
    x = x_ref[...]
    r = x.shape[0]
    bits = jax.lax.bitcast_convert_type(x, jnp.int32)
    key = bits ^ (jax.lax.shift_right_arithmetic(bits, 31) & SIGN_LO)
    row = jax.lax.broadcasted_iota(jnp.int32, (r, N), 0) + i * r
    col = jax.lax.broadcasted_iota(jnp.int32, (r, N), 1)
    is_diag = row == col
    key = jnp.where(is_diag, INT32_MIN, key)

    h16 = jax.lax.shift_right_arithmetic(key, 16).astype(jnp.int16)

    def fold_count(v16):
        w = N
        while w > 128:
            w //= 2
            v16 = v16[:, :w] + v16[:, w:]
        return jnp.sum(v16.astype(jnp.int32), axis=1, keepdims=True)

    def stepA(carry, _):
        lo, hi, chi = carry
        mid = lo + jax.lax.shift_right_logical(hi - lo, 1)
        cnt = fold_count(jnp.where(h16 >= mid.astype(jnp.int16), ONE16, ZERO16))
        ge = cnt >= KOFF
        return (jnp.where(ge, mid, lo), jnp.where(ge, hi, mid),
                jnp.where(ge, chi, cnt)), None

    loA = jnp.full((r, 1), np.int32(-32768))
    hiA = jnp.full((r, 1), np.int32(32768))
    chiA = jnp.zeros((r, 1), jnp.int32)
    (hstar, _, cgt), _ = jax.lax.scan(stepA, (loA, hiA, chiA), None, length=16)
    rstar = KOFF - cgt

    ls = key.astype(jnp.int16) ^ np.int16(-32768)
    inb = jnp.where(h16 == hstar.astype(jnp.int16), ONE16, ZERO16)

    def stepB(carry, _):
        lo, hi = carry
        mid = lo + jax.lax.shift_right_logical(hi - lo, 1)
        cnt = fold_count(jnp.where(ls >= mid.astype(jnp.int16), inb, ZERO16))
        ge = cnt >= rstar
        return (jnp.where(ge, mid, lo), jnp.where(ge, hi, mid)), None

    loB = jnp.full((r, 1), np.int32(-32768))
    hiB = jnp.full((r, 1), np.int32(32768))
    (lstar, _), _ = jax.lax.scan(stepB, (loB, hiB), None, length=16)

    thr = (jax.lax.shift_left(hstar, 16)
           | ((lstar ^ np.int32(0x8000)) & np.int32(0xFFFF)))

    a = jax.nn.sigmoid(x)
    keep = jnp.logical_or(key >= thr, is_diag)
    a = jnp.where(is_diag, jnp.float32(1.0), a)
    o_ref[...] = jnp.where(keep, a, jnp.float32(0.0))


@jax.jit
def kernel(logits):
    grid = (N // BLOCK_ROWS,)
    return pl.pallas_call(
        _block_kernel,
        grid=grid,
        in_specs=[pl.BlockSpec((BLOCK_ROWS, N), lambda i: (i, 0))],
        out_specs=pl.BlockSpec((BLOCK_ROWS, N), lambda i: (i, 0)),
        out_shape=jax.ShapeDtypeStruct((N, N), jnp.float32),
        compiler_params=pltpu.CompilerParams(
            dimension_semantics=("arbitrary",),
        ),
    )(logits)


# speedup vs baseline: 2.6417x; 1.1680x over previous
import jax
import jax.numpy as jnp
import numpy as np
from jax.experimental import pallas as pl
from jax.experimental.pallas import tpu as pltpu

N = 4096
KOFF = 63
BLOCK_ROWS = 512
INT32_MIN = np.int32(-2147483648)
SIGN_LO = np.int32(0x7FFFFFFF)
ONE16 = np.int16(1)
ZERO16 = np.int16(0)


def _block_kernel(x_ref, o_ref):
    i = pl.program_id(0)
    x = x_ref[...]
    r = x.shape[0]
    bits = jax.lax.bitcast_convert_type(x, jnp.int32)
    key = bits ^ (jax.lax.shift_right_arithmetic(bits, 31) & SIGN_LO)
    row = jax.lax.broadcasted_iota(jnp.int32, (r, N), 0) + i * r
    col = jax.lax.broadcasted_iota(jnp.int32, (r, N), 1)
    is_diag = row == col
    key = jnp.where(is_diag, INT32_MIN, key)

    h16 = jax.lax.shift_right_arithmetic(key, 16).astype(jnp.int16)

    def fold_count(v16):
        w = N
        while w > 128:
            w //= 2
            v16 = v16[:, :w] + v16[:, w:]
        return jnp.sum(v16.astype(jnp.int32), axis=1, keepdims=True)

    def stepA(carry, _):
        lo, hi, chi = carry
        mid = lo + jax.lax.shift_right_logical(hi - lo, 1)
        cnt = fold_count(jnp.where(h16 >= mid.astype(jnp.int16), ONE16, ZERO16))
        ge = cnt >= KOFF
        return (jnp.where(ge, mid, lo), jnp.where(ge, hi, mid),
                jnp.where(ge, chi, cnt)), None

    loA = jnp.full((r, 1), np.int32(-32768))
    hiA = jnp.full((r, 1), np.int32(32768))
    chiA = jnp.zeros((r, 1), jnp.int32)
    (hstar, _, cgt), _ = jax.lax.scan(stepA, (loA, hiA, chiA), None,
                                      length=16, unroll=4)
    rstar = KOFF - cgt

    ls = key.astype(jnp.int16) ^ np.int16(-32768)
    inb = jnp.where(h16 == hstar.astype(jnp.int16), ONE16, ZERO16)

    def stepB(carry, _):
        lo, hi = carry
        mid = lo + jax.lax.shift_right_logical(hi - lo, 1)
        cnt = fold_count(jnp.where(ls >= mid.astype(jnp.int16), inb, ZERO16))
        ge = cnt >= rstar
        return (jnp.where(ge, mid, lo), jnp.where(ge, hi, mid)), None

    loB = jnp.full((r, 1), np.int32(-32768))
    hiB = jnp.full((r, 1), np.int32(32768))
    (lstar, _), _ = jax.lax.scan(stepB, (loB, hiB), None, length=16, unroll=4)

    thr = (jax.lax.shift_left(hstar, 16)
           | ((lstar ^ np.int32(0x8000)) & np.int32(0xFFFF)))

    a = jax.nn.sigmoid(x)
    keep = jnp.logical_or(key >= thr, is_diag)
    a = jnp.where(is_diag, jnp.float32(1.0), a)
    o_ref[...] = jnp.where(keep, a, jnp.float32(0.0))


@jax.jit
def kernel(logits):
    grid = (N // BLOCK_ROWS,)
    return pl.pallas_call(
        _block_kernel,
        grid=grid,
        in_specs=[pl.BlockSpec((BLOCK_ROWS, N), lambda i: (i, 0))],
        out_specs=pl.BlockSpec((BLOCK_ROWS, N), lambda i: (i, 0)),
        out_shape=jax.ShapeDtypeStruct((N, N), jnp.float32),
        compiler_params=pltpu.CompilerParams(
            dimension_semantics=("arbitrary",),
        ),
    )(logits)
